# baseline (speedup 1.0000x reference)
import functools

import jax
import jax.numpy as jnp
from jax.experimental import pallas as pl
from jax.experimental.pallas import tpu as pltpu

_CG = 16
_EPS = 1e-3
_NS_ITERS = 10
_NB = 4
_SB = 8


def _stats_kernel(x_ref, p_ref):
    j = pl.program_id(0)
    pp = None
    for k in range(_NB):
        x = x_ref[k]
        ones = jnp.ones((1, x.shape[1]), dtype=x.dtype)
        xa = jnp.concatenate([x, ones], axis=0)
        part = jax.lax.dot_general(
            x, xa, (((1,), (1,)), ((), ())),
            preferred_element_type=jnp.float32)
        pp = part if pp is None else pp + part

    @pl.when(j == 0)
    def _():
        p_ref[...] = pp

    @pl.when(j > 0)
    def _():
        p_ref[...] += pp


def _solve_kernel(p_ref, w_ref, b_ref, wf_ref, off_ref, *, inv_m):
    c = w_ref.shape[0]
    pt = p_ref[...]
    mean = pt[:, c:c + 1] * inv_m
    outer = jax.lax.dot_general(
        mean, mean, (((1,), (1,)), ((), ())),
        preferred_element_type=jnp.float32)
    rows = jax.lax.broadcasted_iota(jnp.int32, (c, c), 0)
    cols = jax.lax.broadcasted_iota(jnp.int32, (c, c), 1)
    blk = (rows // _CG) == (cols // _CG)
    maskf = jnp.where(blk, 1.0, 0.0).astype(jnp.float32)
    eyef = jnp.where(rows == cols, 1.0, 0.0).astype(jnp.float32)
    sigma = (pt[:, :c] * inv_m - outer) * maskf + _EPS * eyef

    rs = jnp.sum(sigma * sigma, axis=1, keepdims=True)
    f2 = jax.lax.dot_general(
        maskf, rs, (((1,), (0,)), ((), ())),
        preferred_element_type=jnp.float32)
    invf = jax.lax.rsqrt(f2)
    y = sigma * invf
    z = eyef
    dn = (((1,), (0,)), ((), ()))
    for _ in range(_NS_ITERS):
        t = 1.5 * eyef - 0.5 * jax.lax.dot_general(
            z, y, dn, preferred_element_type=jnp.float32)
        y = jax.lax.dot_general(y, t, dn, preferred_element_type=jnp.float32)
        z = jax.lax.dot_general(t, z, dn, preferred_element_type=jnp.float32)
    wm = z * jnp.sqrt(invf)
    wf = wm * w_ref[...]
    off = b_ref[...] - jax.lax.dot_general(
        wf, mean, dn, preferred_element_type=jnp.float32)
    wf_ref[...] = wf
    off_ref[...] = off


def _apply_kernel(x_ref, wf_ref, off_ref, o_ref, ibuf, obuf, isems, osems):
    s = pl.program_id(0)
    nsb = pl.num_programs(0)
    base = s * _SB

    @pl.when(s > 0)
    def _():
        for r in range(_SB):
            pltpu.make_async_copy(
                obuf.at[pl.ds(r, 1)], o_ref.at[pl.ds(base - _SB + r, 1)],
                osems.at[r]).wait()

    for r in range(_SB):
        pltpu.make_async_copy(
            x_ref.at[pl.ds(base + r, 1)], ibuf.at[pl.ds(r, 1)],
            isems.at[r]).start()
    for r in range(_SB):
        pltpu.make_async_copy(
            x_ref.at[pl.ds(base + r, 1)], ibuf.at[pl.ds(r, 1)],
            isems.at[r]).wait()

    wf = wf_ref[...]
    off = off_ref[...]
    dn = (((1,), (0,)), ((), ()))
    for r in range(_SB):
        obuf[r] = jax.lax.dot_general(
            wf, ibuf[r], dn, preferred_element_type=jnp.float32) + off
        pltpu.make_async_copy(
            obuf.at[pl.ds(r, 1)], o_ref.at[pl.ds(base + r, 1)],
            osems.at[r]).start()

    @pl.when(s == nsb - 1)
    def _():
        for r in range(_SB):
            pltpu.make_async_copy(
                obuf.at[pl.ds(r, 1)], o_ref.at[pl.ds(base + r, 1)],
                osems.at[r]).wait()


def kernel(X, weight, bias):
    n, c, h, w = X.shape
    hw = h * w
    x3 = X.reshape(n, c, hw)

    p2 = pl.pallas_call(
        _stats_kernel,
        grid=(n // _NB,),
        in_specs=[pl.BlockSpec((_NB, c, hw), lambda j: (j, 0, 0))],
        out_specs=pl.BlockSpec((c, c + 1), lambda j: (0, 0)),
        out_shape=jax.ShapeDtypeStruct((c, c + 1), jnp.float32),
        compiler_params=pltpu.CompilerParams(
            dimension_semantics=("arbitrary",),
            vmem_limit_bytes=56 * 1024 * 1024),
    )(x3)

    wf, off = pl.pallas_call(
        functools.partial(_solve_kernel, inv_m=1.0 / (n * hw)),
        out_shape=(jax.ShapeDtypeStruct((c, c), jnp.float32),
                   jax.ShapeDtypeStruct((c, 1), jnp.float32)),
    )(p2, weight.reshape(c, 1), bias.reshape(c, 1))

    y3 = pl.pallas_call(
        _apply_kernel,
        grid=(n // _SB,),
        in_specs=[pl.BlockSpec(memory_space=pl.ANY),
                  pl.BlockSpec((c, c), lambda s: (0, 0)),
                  pl.BlockSpec((c, 1), lambda s: (0, 0))],
        out_specs=pl.BlockSpec(memory_space=pl.ANY),
        out_shape=jax.ShapeDtypeStruct((n, c, hw), jnp.float32),
        scratch_shapes=[
            pltpu.VMEM((_SB, c, hw), jnp.float32),
            pltpu.VMEM((_SB, c, hw), jnp.float32),
            pltpu.SemaphoreType.DMA((_SB,)),
            pltpu.SemaphoreType.DMA((_SB,)),
        ],
        compiler_params=pltpu.CompilerParams(
            dimension_semantics=("arbitrary",),
            vmem_limit_bytes=56 * 1024 * 1024),
    )(x3, wf, off)

    return y3.reshape(n, c, h, w)


# speedup vs baseline: 1.0193x; 1.0193x over previous
import functools

import jax
import jax.numpy as jnp
from jax.experimental import pallas as pl
from jax.experimental.pallas import tpu as pltpu

_CG = 16
_EPS = 1e-3
_NS_ITERS = 10
_NB = 4
_SLOTS = 12


def _stats_kernel(x_ref, p_ref):
    j = pl.program_id(0)
    pp = None
    for k in range(_NB):
        x = x_ref[k]
        ones = jnp.ones((1, x.shape[1]), dtype=x.dtype)
        xa = jnp.concatenate([x, ones], axis=0)
        part = jax.lax.dot_general(
            x, xa, (((1,), (1,)), ((), ())),
            preferred_element_type=jnp.float32)
        pp = part if pp is None else pp + part

    @pl.when(j == 0)
    def _():
        p_ref[...] = pp

    @pl.when(j > 0)
    def _():
        p_ref[...] += pp


def _solve_kernel(p_ref, w_ref, b_ref, wf_ref, off_ref, *, inv_m):
    c = w_ref.shape[0]
    pt = p_ref[...]
    mean = pt[:, c:c + 1] * inv_m
    outer = jax.lax.dot_general(
        mean, mean, (((1,), (1,)), ((), ())),
        preferred_element_type=jnp.float32)
    rows = jax.lax.broadcasted_iota(jnp.int32, (c, c), 0)
    cols = jax.lax.broadcasted_iota(jnp.int32, (c, c), 1)
    blk = (rows // _CG) == (cols // _CG)
    maskf = jnp.where(blk, 1.0, 0.0).astype(jnp.float32)
    eyef = jnp.where(rows == cols, 1.0, 0.0).astype(jnp.float32)
    sigma = (pt[:, :c] * inv_m - outer) * maskf + _EPS * eyef

    rs = jnp.sum(sigma * sigma, axis=1, keepdims=True)
    f2 = jax.lax.dot_general(
        maskf, rs, (((1,), (0,)), ((), ())),
        preferred_element_type=jnp.float32)
    invf = jax.lax.rsqrt(f2)
    y = sigma * invf
    z = eyef
    dn = (((1,), (0,)), ((), ()))
    for _ in range(_NS_ITERS):
        t = 1.5 * eyef - 0.5 * jax.lax.dot_general(
            z, y, dn, preferred_element_type=jnp.float32)
        y = jax.lax.dot_general(y, t, dn, preferred_element_type=jnp.float32)
        z = jax.lax.dot_general(t, z, dn, preferred_element_type=jnp.float32)
    wm = z * jnp.sqrt(invf)
    wf = wm * w_ref[...]
    off = b_ref[...] - jax.lax.dot_general(
        wf, mean, dn, preferred_element_type=jnp.float32)
    wf_ref[...] = wf
    off_ref[...] = off


def _apply_kernel(x_ref, wf_ref, off_ref, o_ref, buf, sems):
    i = pl.program_id(0)
    n = pl.num_programs(0)
    slot = jax.lax.rem(i, _SLOTS)

    @pl.when(i >= _SLOTS)
    def _():
        pltpu.make_async_copy(
            buf.at[pl.ds(slot, 1)], o_ref.at[pl.ds(i - _SLOTS, 1)],
            sems.at[slot]).wait()

    dn = (((1,), (0,)), ((), ()))
    buf[slot] = jax.lax.dot_general(
        wf_ref[...], x_ref[0], dn,
        preferred_element_type=jnp.float32) + off_ref[...]
    pltpu.make_async_copy(
        buf.at[pl.ds(slot, 1)], o_ref.at[pl.ds(i, 1)], sems.at[slot]).start()

    @pl.when(i == n - 1)
    def _():
        for k in range(_SLOTS):
            s = jax.lax.rem(i + 1 + k, _SLOTS)
            pltpu.make_async_copy(
                buf.at[pl.ds(s, 1)], o_ref.at[pl.ds(i, 1)], sems.at[s]).wait()


def kernel(X, weight, bias):
    n, c, h, w = X.shape
    hw = h * w
    x3 = X.reshape(n, c, hw)

    p2 = pl.pallas_call(
        _stats_kernel,
        grid=(n // _NB,),
        in_specs=[pl.BlockSpec((_NB, c, hw), lambda j: (j, 0, 0))],
        out_specs=pl.BlockSpec((c, c + 1), lambda j: (0, 0)),
        out_shape=jax.ShapeDtypeStruct((c, c + 1), jnp.float32),
        compiler_params=pltpu.CompilerParams(
            dimension_semantics=("arbitrary",),
            vmem_limit_bytes=56 * 1024 * 1024),
    )(x3)

    wf, off = pl.pallas_call(
        functools.partial(_solve_kernel, inv_m=1.0 / (n * hw)),
        out_shape=(jax.ShapeDtypeStruct((c, c), jnp.float32),
                   jax.ShapeDtypeStruct((c, 1), jnp.float32)),
    )(p2, weight.reshape(c, 1), bias.reshape(c, 1))

    y3 = pl.pallas_call(
        _apply_kernel,
        grid=(n,),
        in_specs=[pl.BlockSpec((1, c, hw), lambda i: (i, 0, 0)),
                  pl.BlockSpec((c, c), lambda i: (0, 0)),
                  pl.BlockSpec((c, 1), lambda i: (0, 0))],
        out_specs=pl.BlockSpec(memory_space=pl.ANY),
        out_shape=jax.ShapeDtypeStruct((n, c, hw), jnp.float32),
        scratch_shapes=[
            pltpu.VMEM((_SLOTS, c, hw), jnp.float32),
            pltpu.SemaphoreType.DMA((_SLOTS,)),
        ],
        compiler_params=pltpu.CompilerParams(
            dimension_semantics=("arbitrary",),
            vmem_limit_bytes=56 * 1024 * 1024),
    )(x3, wf, off)

    return y3.reshape(n, c, h, w)


# speedup vs baseline: 1.0288x; 1.0093x over previous
import functools

import jax
import jax.numpy as jnp
from jax.experimental import pallas as pl
from jax.experimental.pallas import tpu as pltpu

_CG = 16
_EPS = 1e-3
_NS_ITERS = 10
_NB = 4
_SLOTS = 8
_AB = 4


def _stats_kernel(x_ref, p_ref):
    j = pl.program_id(0)
    pp = None
    for k in range(_NB):
        x = x_ref[k]
        ones = jnp.ones((1, x.shape[1]), dtype=x.dtype)
        xa = jnp.concatenate([x, ones], axis=0)
        part = jax.lax.dot_general(
            x, xa, (((1,), (1,)), ((), ())),
            preferred_element_type=jnp.float32)
        pp = part if pp is None else pp + part

    @pl.when(j == 0)
    def _():
        p_ref[...] = pp

    @pl.when(j > 0)
    def _():
        p_ref[...] += pp


def _solve_kernel(p_ref, w_ref, b_ref, wf_ref, off_ref, *, inv_m):
    c = w_ref.shape[0]
    pt = p_ref[...]
    mean = pt[:, c:c + 1] * inv_m
    outer = jax.lax.dot_general(
        mean, mean, (((1,), (1,)), ((), ())),
        preferred_element_type=jnp.float32)
    rows = jax.lax.broadcasted_iota(jnp.int32, (c, c), 0)
    cols = jax.lax.broadcasted_iota(jnp.int32, (c, c), 1)
    blk = (rows // _CG) == (cols // _CG)
    maskf = jnp.where(blk, 1.0, 0.0).astype(jnp.float32)
    eyef = jnp.where(rows == cols, 1.0, 0.0).astype(jnp.float32)
    sigma = (pt[:, :c] * inv_m - outer) * maskf + _EPS * eyef

    rs = jnp.sum(sigma * sigma, axis=1, keepdims=True)
    f2 = jax.lax.dot_general(
        maskf, rs, (((1,), (0,)), ((), ())),
        preferred_element_type=jnp.float32)
    invf = jax.lax.rsqrt(f2)
    y = sigma * invf
    z = eyef
    dn = (((1,), (0,)), ((), ()))
    for _ in range(_NS_ITERS):
        t = 1.5 * eyef - 0.5 * jax.lax.dot_general(
            z, y, dn, preferred_element_type=jnp.float32)
        y = jax.lax.dot_general(y, t, dn, preferred_element_type=jnp.float32)
        z = jax.lax.dot_general(t, z, dn, preferred_element_type=jnp.float32)
    wm = z * jnp.sqrt(invf)
    wf = wm * w_ref[...]
    off = b_ref[...] - jax.lax.dot_general(
        wf, mean, dn, preferred_element_type=jnp.float32)
    wf_ref[...] = wf
    off_ref[...] = off


def _apply_kernel(x_ref, wf_ref, off_ref, o_ref, buf, sems):
    i = pl.program_id(0)
    n = pl.num_programs(0)
    half = jax.lax.rem(i, 2) * _AB

    @pl.when(i >= 2)
    def _():
        for r in range(_AB):
            pltpu.make_async_copy(
                buf.at[pl.ds(half + r, 1)],
                o_ref.at[pl.ds((i - 2) * _AB + r, 1)],
                sems.at[half + r]).wait()

    wf = wf_ref[...]
    off = off_ref[...]
    dn = (((1,), (0,)), ((), ()))
    for r in range(_AB):
        buf[half + r] = jax.lax.dot_general(
            wf, x_ref[r], dn, preferred_element_type=jnp.float32) + off
        pltpu.make_async_copy(
            buf.at[pl.ds(half + r, 1)], o_ref.at[pl.ds(i * _AB + r, 1)],
            sems.at[half + r]).start()

    @pl.when(i == n - 1)
    def _():
        for k in range(_SLOTS):
            pltpu.make_async_copy(
                buf.at[pl.ds(k, 1)], o_ref.at[pl.ds(i * _AB, 1)],
                sems.at[k]).wait()


def kernel(X, weight, bias):
    n, c, h, w = X.shape
    hw = h * w
    x3 = X.reshape(n, c, hw)

    p2 = pl.pallas_call(
        _stats_kernel,
        grid=(n // _NB,),
        in_specs=[pl.BlockSpec((_NB, c, hw), lambda j: (j, 0, 0))],
        out_specs=pl.BlockSpec((c, c + 1), lambda j: (0, 0)),
        out_shape=jax.ShapeDtypeStruct((c, c + 1), jnp.float32),
        compiler_params=pltpu.CompilerParams(
            dimension_semantics=("arbitrary",),
            vmem_limit_bytes=56 * 1024 * 1024),
    )(x3)

    wf, off = pl.pallas_call(
        functools.partial(_solve_kernel, inv_m=1.0 / (n * hw)),
        out_shape=(jax.ShapeDtypeStruct((c, c), jnp.float32),
                   jax.ShapeDtypeStruct((c, 1), jnp.float32)),
    )(p2, weight.reshape(c, 1), bias.reshape(c, 1))

    y3 = pl.pallas_call(
        _apply_kernel,
        grid=(n // _AB,),
        in_specs=[pl.BlockSpec((_AB, c, hw), lambda i: (i, 0, 0)),
                  pl.BlockSpec((c, c), lambda i: (0, 0)),
                  pl.BlockSpec((c, 1), lambda i: (0, 0))],
        out_specs=pl.BlockSpec(memory_space=pl.ANY),
        out_shape=jax.ShapeDtypeStruct((n, c, hw), jnp.float32),
        scratch_shapes=[
            pltpu.VMEM((_SLOTS, c, hw), jnp.float32),
            pltpu.SemaphoreType.DMA((_SLOTS,)),
        ],
        compiler_params=pltpu.CompilerParams(
            dimension_semantics=("arbitrary",),
            vmem_limit_bytes=56 * 1024 * 1024),
    )(x3, wf, off)

    return y3.reshape(n, c, h, w)


# speedup vs baseline: 2.9516x; 2.8690x over previous
import functools

import jax
import jax.numpy as jnp
from jax.experimental import pallas as pl
from jax.experimental.pallas import tpu as pltpu

_CG = 16
_EPS = 1e-3
_NS_ITERS = 10
_NB = 4


def _stats_kernel(x_ref, p_ref):
    j = pl.program_id(0)
    pp = None
    for k in range(_NB):
        x = x_ref[k]
        ones = jnp.ones((x.shape[0], 1), dtype=x.dtype)
        xa = jnp.concatenate([x, ones], axis=1)
        part = jax.lax.dot_general(
            x, xa, (((0,), (0,)), ((), ())),
            preferred_element_type=jnp.float32)
        pp = part if pp is None else pp + part

    @pl.when(j == 0)
    def _():
        p_ref[...] = pp

    @pl.when(j > 0)
    def _():
        p_ref[...] += pp


def _solve_kernel(p_ref, w_ref, b_ref, wf_ref, off_ref, *, inv_m):
    c = w_ref.shape[0]
    pt = p_ref[...]
    mean = pt[:, c:c + 1] * inv_m
    outer = jax.lax.dot_general(
        mean, mean, (((1,), (1,)), ((), ())),
        preferred_element_type=jnp.float32)
    rows = jax.lax.broadcasted_iota(jnp.int32, (c, c), 0)
    cols = jax.lax.broadcasted_iota(jnp.int32, (c, c), 1)
    blk = (rows // _CG) == (cols // _CG)
    maskf = jnp.where(blk, 1.0, 0.0).astype(jnp.float32)
    eyef = jnp.where(rows == cols, 1.0, 0.0).astype(jnp.float32)
    sigma = (pt[:, :c] * inv_m - outer) * maskf + _EPS * eyef

    rs = jnp.sum(sigma * sigma, axis=1, keepdims=True)
    f2 = jax.lax.dot_general(
        maskf, rs, (((1,), (0,)), ((), ())),
        preferred_element_type=jnp.float32)
    invf = jax.lax.rsqrt(f2)
    y = sigma * invf
    z = eyef
    dn = (((1,), (0,)), ((), ()))
    for _ in range(_NS_ITERS):
        t = 1.5 * eyef - 0.5 * jax.lax.dot_general(
            z, y, dn, preferred_element_type=jnp.float32)
        y = jax.lax.dot_general(y, t, dn, preferred_element_type=jnp.float32)
        z = jax.lax.dot_general(t, z, dn, preferred_element_type=jnp.float32)
    wm = z * jnp.sqrt(invf)
    wf = wm * w_ref[...]
    off = b_ref[...] - jax.lax.dot_general(
        mean, wf, (((0,), (1,)), ((), ())),
        preferred_element_type=jnp.float32)
    wf_ref[...] = wf
    off_ref[...] = off


def _apply_kernel(x_ref, wf_ref, off_ref, o_ref):
    wf = wf_ref[...]
    off = off_ref[...]
    dn = (((1,), (1,)), ((), ()))
    for k in range(_NB):
        o_ref[k] = jax.lax.dot_general(
            x_ref[k], wf, dn, preferred_element_type=jnp.float32) + off


def kernel(X, weight, bias):
    n, c, h, w = X.shape
    hw = h * w
    x2 = jnp.transpose(X, (0, 2, 3, 1)).reshape(n, hw, c)

    p2 = pl.pallas_call(
        _stats_kernel,
        grid=(n // _NB,),
        in_specs=[pl.BlockSpec((_NB, hw, c), lambda j: (j, 0, 0))],
        out_specs=pl.BlockSpec((c, c + 1), lambda j: (0, 0)),
        out_shape=jax.ShapeDtypeStruct((c, c + 1), jnp.float32),
        compiler_params=pltpu.CompilerParams(
            dimension_semantics=("arbitrary",),
            vmem_limit_bytes=56 * 1024 * 1024),
    )(x2)

    wf, off = pl.pallas_call(
        functools.partial(_solve_kernel, inv_m=1.0 / (n * hw)),
        out_shape=(jax.ShapeDtypeStruct((c, c), jnp.float32),
                   jax.ShapeDtypeStruct((1, c), jnp.float32)),
    )(p2, weight.reshape(c, 1), bias.reshape(1, c))

    y2 = pl.pallas_call(
        _apply_kernel,
        grid=(n // _NB,),
        in_specs=[pl.BlockSpec((_NB, hw, c), lambda j: (j, 0, 0)),
                  pl.BlockSpec((c, c), lambda j: (0, 0)),
                  pl.BlockSpec((1, c), lambda j: (0, 0))],
        out_specs=pl.BlockSpec((_NB, hw, c), lambda j: (j, 0, 0)),
        out_shape=jax.ShapeDtypeStruct((n, hw, c), jnp.float32),
        compiler_params=pltpu.CompilerParams(
            dimension_semantics=("arbitrary",),
            vmem_limit_bytes=56 * 1024 * 1024),
    )(x2, wf, off)

    return jnp.transpose(y2.reshape(n, h, w, c), (0, 3, 1, 2))
